# baseline (speedup 1.0000x reference)
import functools

import jax
import jax.numpy as jnp
from jax import lax
from jax.experimental import pallas as pl
from jax.experimental.pallas import tpu as pltpu
from jax.experimental.pallas import tpu_sc as plsc

NUM_ROWS = 16384
NUM_COLS = 20
EMBED_DIM = 32
PADDED = 128
B = NUM_ROWS * NUM_COLS

NC = 2
NS = 16
NW = NC * NS
B_PER_W = B // NW
CHUNK = 640
N_CHUNKS = B_PER_W // CHUNK
NBUF = 1


@functools.partial(
    pl.kernel,
    mesh=plsc.VectorSubcoreMesh(core_axis_name="c", subcore_axis_name="s"),
    out_type=(
        jax.ShapeDtypeStruct((B, EMBED_DIM), jnp.float32),
        jax.ShapeDtypeStruct((B, EMBED_DIM), jnp.float32),
    ),
    scratch_types=[
        pltpu.VMEM((2, B_PER_W), jnp.int32),
        pltpu.VMEM((NBUF, CHUNK, PADDED), jnp.float32),
        pltpu.SemaphoreType.DMA((2,)),
        pltpu.SemaphoreType.DMA((NBUF,)),
        pltpu.SemaphoreType.DMA((NBUF,)),
    ],
    compiler_params=pltpu.CompilerParams(use_tc_tiling_on_sc=False),
)
def _double_gather(w_sr, w_tg, idx_sr, idx_tg, o_sr, o_tg,
                   idx_v, rows_v, isem, gsem, wsem):
    wid = lax.axis_index("s") * NC + lax.axis_index("c")
    base = wid * B_PER_W
    icopy = [
        pltpu.async_copy(idx_sr.at[pl.ds(base, B_PER_W)], idx_v.at[0], isem.at[0]),
        pltpu.async_copy(idx_tg.at[pl.ds(base, B_PER_W)], idx_v.at[1], isem.at[1]),
    ]
    wcopy = None
    for t, (w, o) in enumerate(((w_sr, o_sr), (w_tg, o_tg))):
        icopy[t].wait()
        for g in range(N_CHUNKS):
            if wcopy is not None:
                wcopy.wait()
            gcopy = pltpu.async_copy(
                w.at[idx_v.at[t].at[pl.ds(g * CHUNK, CHUNK)]],
                rows_v.at[0], gsem.at[0])
            gcopy.wait()
            wcopy = pltpu.async_copy(
                rows_v.at[0, :, pl.ds(0, EMBED_DIM)],
                o.at[pl.ds(base + g * CHUNK, CHUNK)], wsem.at[0])
    wcopy.wait()


def kernel(sr_data, tg_data, W_sr, W_tg):
    idx_sr = sr_data.reshape(B)
    idx_tg = tg_data.reshape(B)
    Wp_sr = jnp.pad(W_sr, ((0, 0), (0, PADDED - EMBED_DIM)))
    Wp_tg = jnp.pad(W_tg, ((0, 0), (0, PADDED - EMBED_DIM)))
    o_sr, o_tg = _double_gather(Wp_sr, Wp_tg, idx_sr, idx_tg)
    return (
        o_sr.reshape(NUM_ROWS, NUM_COLS, EMBED_DIM),
        o_tg.reshape(NUM_ROWS, NUM_COLS, EMBED_DIM),
    )
